# baseline (speedup 1.0000x reference)
import functools

import jax
import jax.numpy as jnp
from jax import lax
from jax.experimental import pallas as pl
from jax.experimental.pallas import tpu as pltpu
from jax.experimental.pallas import tpu_sc as plsc

B = 16384
D = 64
TAB_R = 500000

NC = 2
NS = 16
NW = NC * NS
B_PER_W = B // NW
CHUNK = 128
NCHUNK = B_PER_W // CHUNK
L = 16


def _select_half(idx_v, off, pairs_v, out_v):
    iota = lax.iota(jnp.int32, L)

    def group_body(g, _):
        rowvec = g * L + iota
        u16 = idx_v[pl.ds(off + g * L, L)]
        lvec = (u16 & 1) * D

        def col_body(c, _):
            cvec = jnp.full((L,), c, dtype=jnp.int32)
            x = plsc.load_gather(pairs_v, [rowvec, lvec + cvec])
            plsc.store_scatter(out_v, [rowvec, cvec], x)
            return 0

        return lax.fori_loop(0, D, col_body, 0, unroll=4)

    lax.fori_loop(0, CHUNK // L, group_body, 0)


def _gather_body(tab_hbm, idx_q_hbm, idx_a_hbm, q_out_hbm, a_out_hbm,
                 idx_q_v, idx_a_v, b_q_v, b_a_v, p_v, o_v, sem0, sem1):
    wid = lax.axis_index("s") * NC + lax.axis_index("c")
    base = wid * B_PER_W
    sems = (sem0, sem1)

    pltpu.sync_copy(idx_q_hbm.at[pl.ds(base, B_PER_W)], idx_q_v)
    pltpu.sync_copy(idx_a_hbm.at[pl.ds(base, B_PER_W)], idx_a_v)

    for j in range(B_PER_W // L):
        sl = pl.ds(j * L, L)
        b_q_v[sl] = lax.shift_right_logical(idx_q_v[sl], 1)
        b_a_v[sl] = lax.shift_right_logical(idx_a_v[sl], 1)

    def fire(j):
        buf = j % 2
        sl = pl.ds(j * CHUNK, CHUNK)
        cq = pltpu.async_copy(
            tab_hbm.at[b_q_v.at[sl]], p_v.at[buf, 0], sems[buf])
        ca = pltpu.async_copy(
            tab_hbm.at[b_a_v.at[sl]], p_v.at[buf, 1], sems[buf])
        return (cq, ca)

    inflight = fire(0)
    for j in range(NCHUNK):
        cq, ca = inflight
        nxt = fire(j + 1) if j + 1 < NCHUNK else None
        cq.wait()
        ca.wait()
        buf = j % 2
        out_sl = pl.ds(base + j * CHUNK, CHUNK)
        _select_half(idx_q_v, j * CHUNK, p_v.at[buf, 0], o_v.at[0])
        pltpu.sync_copy(o_v.at[0], q_out_hbm.at[out_sl])
        _select_half(idx_a_v, j * CHUNK, p_v.at[buf, 1], o_v.at[1])
        pltpu.sync_copy(o_v.at[1], a_out_hbm.at[out_sl])
        inflight = nxt


@jax.jit
def _gather2(table, idx_q, idx_a):
    run = functools.partial(
        pl.kernel,
        out_type=(
            jax.ShapeDtypeStruct((B, D), jnp.float32),
            jax.ShapeDtypeStruct((B, D), jnp.float32),
        ),
        mesh=plsc.VectorSubcoreMesh(core_axis_name="c", subcore_axis_name="s"),
        scratch_types=[
            pltpu.VMEM((B_PER_W,), jnp.int32),
            pltpu.VMEM((B_PER_W,), jnp.int32),
            pltpu.VMEM((B_PER_W,), jnp.int32),
            pltpu.VMEM((B_PER_W,), jnp.int32),
            pltpu.VMEM((2, 2, CHUNK, 2 * D), jnp.float32),
            pltpu.VMEM((2, CHUNK, D), jnp.float32),
            pltpu.SemaphoreType.DMA,
            pltpu.SemaphoreType.DMA,
        ],
        compiler_params=pltpu.CompilerParams(
            use_tc_tiling_on_sc=True, needs_layout_passes=False),
    )(_gather_body)
    return run(table, idx_q, idx_a)


def kernel(cat_q, num_q, cat_qu, num_qu, cat_au, num_au, emb_user):
    idx_q = cat_qu.reshape(B)
    idx_a = cat_au.reshape(B)
    tab = emb_user.reshape(TAB_R, 2 * D)
    return _gather2(tab, idx_q, idx_a)


# speedup vs baseline: 1.2347x; 1.2347x over previous
import functools

import jax
import jax.numpy as jnp
from jax import lax
from jax.experimental import pallas as pl
from jax.experimental.pallas import tpu as pltpu
from jax.experimental.pallas import tpu_sc as plsc

B = 16384
V = 1000000
D = 64

NC = 2
NS = 16
NW = NC * NS
B_PER_W = B // NW
CHUNK = 128
NCHUNK = B_PER_W // CHUNK
L = 16


def _gather_body(tab_hbm, idx_q_hbm, idx_a_hbm, q_out_hbm, a_out_hbm,
                 idx_q_v, idx_a_v, p_v, sem0, sem1):
    wid = lax.axis_index("s") * NC + lax.axis_index("c")
    base = wid * B_PER_W
    sems = (sem0, sem1)
    idxs = (idx_q_v, idx_a_v)
    outs = (q_out_hbm, a_out_hbm)

    pltpu.sync_copy(idx_q_hbm.at[pl.ds(base, B_PER_W)], idx_q_v)
    pltpu.sync_copy(idx_a_hbm.at[pl.ds(base, B_PER_W)], idx_a_v)

    NU = 2 * NCHUNK

    def fire(u):
        s, j = u & 1, u >> 1
        buf = u % 2
        sl = pl.ds(j * CHUNK, CHUNK)
        return pltpu.async_copy(
            tab_hbm.at[idxs[s].at[sl]], p_v.at[buf], sems[buf])

    inflight = [fire(0), fire(1)]
    for u in range(NU):
        s, j = u & 1, u >> 1
        inflight[u % 2].wait()
        pltpu.sync_copy(p_v.at[u % 2],
                        outs[s].at[pl.ds(base + j * CHUNK, CHUNK)])
        if u + 2 < NU:
            inflight[u % 2] = fire(u + 2)


@jax.jit
def _gather2(table, idx_q, idx_a):
    run = functools.partial(
        pl.kernel,
        out_type=(
            jax.ShapeDtypeStruct((B, 2 * D), jnp.float32),
            jax.ShapeDtypeStruct((B, 2 * D), jnp.float32),
        ),
        mesh=plsc.VectorSubcoreMesh(core_axis_name="c", subcore_axis_name="s"),
        scratch_types=[
            pltpu.VMEM((B_PER_W,), jnp.int32),
            pltpu.VMEM((B_PER_W,), jnp.int32),
            pltpu.VMEM((2, CHUNK, 2 * D), jnp.float32),
            pltpu.SemaphoreType.DMA,
            pltpu.SemaphoreType.DMA,
        ],
        compiler_params=pltpu.CompilerParams(
            use_tc_tiling_on_sc=True, needs_layout_passes=False),
    )(_gather_body)
    return run(table, idx_q, idx_a)


def kernel(cat_q, num_q, cat_qu, num_qu, cat_au, num_au, emb_user):
    idx_q = cat_qu.reshape(B)
    idx_a = cat_au.reshape(B)
    tab = jnp.pad(emb_user, ((0, 0), (0, D)))
    q_full, a_full = _gather2(tab, idx_q, idx_a)
    return (q_full[:, :D], a_full[:, :D])
